# baseline (speedup 1.0000x reference)
import functools

import jax
import jax.numpy as jnp
from jax import lax
from jax.experimental import pallas as pl
from jax.experimental.pallas import tpu as pltpu
from jax.experimental.pallas import tpu_sc as plsc

B = 16384
D = 64
NC = 2
NS = 16
NW = NC * NS
BPW = B // NW
CHUNK = 128
NCHUNK = BPW // CHUNK
PAIRW = 2 * D
TBLK = 512


def _tc_pair_rows(t_ref, ee_ref, eo_ref, out_ref):
    a = t_ref[...]
    left = lax.dot_general(ee_ref[...], a, (((0,), (1,)), ((), ())),
                           precision=lax.Precision.HIGHEST)
    right = lax.dot_general(eo_ref[...], a, (((0,), (1,)), ((), ())),
                            precision=lax.Precision.HIGHEST)
    out_ref[...] = jnp.concatenate([left, right], axis=1)


def _to_pair_rows(table_t, e_even, e_odd):
    nfeat, v = table_t.shape
    grid = (v + TBLK - 1) // TBLK
    return pl.pallas_call(
        _tc_pair_rows,
        grid=(grid,),
        in_specs=[
            pl.BlockSpec((nfeat, TBLK), lambda j: (0, j)),
            pl.BlockSpec((TBLK, TBLK // 2), lambda j: (0, 0)),
            pl.BlockSpec((TBLK, TBLK // 2), lambda j: (0, 0)),
        ],
        out_specs=pl.BlockSpec((TBLK // 2, 2 * nfeat), lambda j: (j, 0)),
        out_shape=jax.ShapeDtypeStruct((v // 2, 2 * nfeat), jnp.float32),
    )(table_t, e_even, e_odd)


def _sc_cf_dot(xu_hbm, xi_hbm, user_hbm, item_hbm, out_hbm,
               idx_u, idx_i, off_u, off_i, buf_u, buf_i, out_v,
               sem_u, sem_i):
    cid = lax.axis_index("c")
    sid = lax.axis_index("s")
    wid = sid * NC + cid
    base = wid * BPW

    pltpu.sync_copy(xu_hbm.at[pl.ds(base, BPW)], idx_u)
    pltpu.sync_copy(xi_hbm.at[pl.ds(base, BPW)], idx_i)

    iota16 = lax.iota(jnp.int32, 16)
    for g in range(BPW // 16):
        raw_u = idx_u[pl.ds(g * 16, 16)]
        raw_i = idx_i[pl.ds(g * 16, 16)]
        idx_u[pl.ds(g * 16, 16)] = lax.shift_right_logical(raw_u, 1)
        idx_i[pl.ds(g * 16, 16)] = lax.shift_right_logical(raw_i, 1)
        off_u[pl.ds(g * 16, 16)] = (raw_u & 1) * D
        off_i[pl.ds(g * 16, 16)] = (raw_i & 1) * D

    def fire(c, slot):
        cu = pltpu.async_copy(
            user_hbm.at[idx_u.at[pl.ds(c * CHUNK, CHUNK)]],
            buf_u.at[slot], sem_u)
        ci = pltpu.async_copy(
            item_hbm.at[idx_i.at[pl.ds(c * CHUNK, CHUNK)]],
            buf_i.at[slot], sem_i)
        return cu, ci

    lane15 = iota16 == 15
    inflight = fire(0, 0)
    for c in range(NCHUNK):
        if c + 1 < NCHUNK:
            nxt = fire(c + 1, (c + 1) % 2)
        for cp in inflight:
            cp.wait()
        slot = c % 2

        def body(rr, carry, c=c, slot=slot):
            su = pl.multiple_of(off_u[pl.ds(c * CHUNK + rr, 16)][0], 16)
            si = pl.multiple_of(off_i[pl.ds(c * CHUNK + rr, 16)][0], 16)
            acc = (buf_u[slot, rr, pl.ds(su, 16)]
                   * buf_i[slot, rr, pl.ds(si, 16)])
            for k in range(1, D // 16):
                acc = acc + (buf_u[slot, rr, pl.ds(su + k * 16, 16)]
                             * buf_i[slot, rr, pl.ds(si + k * 16, 16)])
            cs = plsc.cumsum(acc)
            plsc.store_scatter(
                out_v, [jnp.full((16,), c * CHUNK, jnp.int32) + rr], cs,
                mask=lane15)
            return carry
        lax.fori_loop(0, CHUNK, body, 0)
        if c + 1 < NCHUNK:
            inflight = nxt

    pltpu.sync_copy(out_v, out_hbm.at[pl.ds(base, BPW)])


@jax.jit
def kernel(x, user_emb, item_emb):
    mesh = plsc.VectorSubcoreMesh(core_axis_name="c", subcore_axis_name="s")
    f = functools.partial(
        pl.kernel,
        mesh=mesh,
        out_type=jax.ShapeDtypeStruct((B,), jnp.float32),
        scratch_types=[
            pltpu.VMEM((BPW,), jnp.int32),
            pltpu.VMEM((BPW,), jnp.int32),
            pltpu.VMEM((BPW + 16,), jnp.int32),
            pltpu.VMEM((BPW + 16,), jnp.int32),
            pltpu.VMEM((2, CHUNK, PAIRW), jnp.float32),
            pltpu.VMEM((2, CHUNK, PAIRW), jnp.float32),
            pltpu.VMEM((BPW,), jnp.float32),
            pltpu.SemaphoreType.DMA,
            pltpu.SemaphoreType.DMA,
        ],
        compiler_params=pltpu.CompilerParams(
            needs_layout_passes=False, use_tc_tiling_on_sc=True),
    )(_sc_cf_dot)
    x32 = x.astype(jnp.int32)
    c_idx = lax.broadcasted_iota(jnp.int32, (TBLK, TBLK // 2), 0)
    i_idx = lax.broadcasted_iota(jnp.int32, (TBLK, TBLK // 2), 1)
    e_even = (c_idx == 2 * i_idx).astype(jnp.float32)
    e_odd = (c_idx == 2 * i_idx + 1).astype(jnp.float32)
    return f(x32[:, 0], x32[:, 1],
             _to_pair_rows(user_emb.T, e_even, e_odd),
             _to_pair_rows(item_emb.T, e_even, e_odd))


# speedup vs baseline: 3.2988x; 3.2988x over previous
import functools

import jax
import jax.numpy as jnp
from jax import lax
from jax.experimental import pallas as pl
from jax.experimental.pallas import tpu as pltpu
from jax.experimental.pallas import tpu_sc as plsc

B = 16384
D = 64
NC = 2
NS = 16
NW = NC * NS
BPW = B // NW
CHUNK = 128
NCHUNK = BPW // CHUNK

_MESH = dict(
    mesh=plsc.VectorSubcoreMesh(core_axis_name="c", subcore_axis_name="s"),
    compiler_params=pltpu.CompilerParams(
        needs_layout_passes=False, use_tc_tiling_on_sc=False),
)


def _deinterleave(x_hbm, x_v, idx, base, col):
    pltpu.sync_copy(x_hbm.at[pl.ds(base * 2, BPW * 2)], x_v)
    iota16 = lax.iota(jnp.int32, 16)
    for g in range(BPW // 16):
        pos16 = (iota16 + (g * 16)) * 2 + col
        c = (g * 16) // CHUNK
        off = (g * 16) % CHUNK
        idx[c, pl.ds(off, 16)] = plsc.load_gather(x_v, [pos16])


def _gather_user(x_hbm, user_hbm, u_hbm, x_v, idx_u, rows_u, sem):
    wid = lax.axis_index("s") * NC + lax.axis_index("c")
    base = wid * BPW
    _deinterleave(x_hbm, x_v, idx_u, base, 0)
    copies = [
        pltpu.async_copy(user_hbm.at[idx_u.at[c]],
                         rows_u.at[pl.ds(c * CHUNK, CHUNK), :], sem)
        for c in range(NCHUNK)
    ]
    for cp in copies:
        cp.wait()
    pltpu.sync_copy(rows_u, u_hbm.at[pl.ds(base, BPW), :])


def _gather_item_dot(x_hbm, item_hbm, u_rows_hbm, out_hbm,
                     x_v, idx_i, rows_u, rows_i, out_v, sem, sem_u):
    wid = lax.axis_index("s") * NC + lax.axis_index("c")
    base = wid * BPW
    cp_u = pltpu.async_copy(u_rows_hbm.at[pl.ds(base, BPW), :], rows_u, sem_u)
    _deinterleave(x_hbm, x_v, idx_i, base, 1)
    copies = [
        pltpu.async_copy(item_hbm.at[idx_i.at[c]],
                         rows_i.at[pl.ds(c * CHUNK, CHUNK), :], sem)
        for c in range(NCHUNK)
    ]
    cp_u.wait()
    for cp in copies:
        cp.wait()

    iota16 = lax.iota(jnp.int32, 16)
    lane15 = iota16 == 15

    def body(r, carry):
        acc = rows_u[r, pl.ds(0, 16)] * rows_i[r, pl.ds(0, 16)]
        for k in range(1, D // 16):
            acc = acc + rows_u[r, pl.ds(k * 16, 16)] * rows_i[r, pl.ds(k * 16, 16)]
        cs = plsc.cumsum(acc)
        plsc.store_scatter(out_v, [jnp.full((16,), r, jnp.int32)], cs,
                           mask=lane15)
        return carry
    lax.fori_loop(0, BPW, body, 0)

    pltpu.sync_copy(out_v, out_hbm.at[pl.ds(base, BPW)])


@jax.jit
def kernel(x, user_emb, item_emb):
    xf = x.astype(jnp.int32).reshape(-1)
    ka = functools.partial(
        pl.kernel,
        out_type=jax.ShapeDtypeStruct((B, D), jnp.float32),
        scratch_types=[
            pltpu.VMEM((BPW * 2,), jnp.int32),
            pltpu.VMEM((NCHUNK, CHUNK), jnp.int32),
            pltpu.VMEM((BPW, D), jnp.float32),
            pltpu.SemaphoreType.DMA,
        ],
        **_MESH,
    )(_gather_user)
    u_rows = ka(xf, user_emb)
    kb = functools.partial(
        pl.kernel,
        out_type=jax.ShapeDtypeStruct((B,), jnp.float32),
        scratch_types=[
            pltpu.VMEM((BPW * 2,), jnp.int32),
            pltpu.VMEM((NCHUNK, CHUNK), jnp.int32),
            pltpu.VMEM((BPW, D), jnp.float32),
            pltpu.VMEM((BPW, D), jnp.float32),
            pltpu.VMEM((BPW,), jnp.float32),
            pltpu.SemaphoreType.DMA,
            pltpu.SemaphoreType.DMA,
        ],
        **_MESH,
    )(_gather_item_dot)
    return kb(xf, item_emb, u_rows)
